# baseline (speedup 1.0000x reference)
import jax
import jax.numpy as jnp
from jax.experimental import pallas as pl
from jax.experimental.pallas import tpu as pltpu

N = 2048
D = 2048
H = 16384
K = 64

_BH_ENC = 512
_BN_SEL = 128
_BH_DEC = 512


def _encode_kernel(x_ref, w_ref, pb_ref, lb_ref, out_ref):
    xc = x_ref[...] - pb_ref[...]
    pa = jax.lax.dot_general(
        xc, w_ref[...], (((1,), (1,)), ((), ())),
        preferred_element_type=jnp.float32) + lb_ref[...]
    b = jax.lax.bitcast_convert_type(pa, jnp.int32)
    out_ref[...] = jnp.where(b >= 0, b, b ^ jnp.int32(0x7FFFFFFF))


def _select_kernel(keys_ref, feat_ref, lo_ref, hi_ref, done_ref):
    keys = keys_ref[...]
    lo0 = jnp.min(keys, axis=1, keepdims=True)
    hi0 = jnp.max(keys, axis=1, keepdims=True) + 1
    done0 = (lo0 + 1 >= hi0).astype(jnp.int32)
    lo_ref[...] = lo0
    hi_ref[...] = hi0
    done_ref[...] = done0

    def cond(n_active):
        return n_active > 0

    def body(_):
        lo = lo_ref[...]
        hi = hi_ref[...]
        done = done_ref[...] > 0
        mid = (lo >> 1) + (hi >> 1) + (lo & hi & 1)
        cnt = jnp.sum((keys >= mid).astype(jnp.int32), axis=1, keepdims=True)
        ok = cnt >= K
        nlo = jnp.where(done, lo, jnp.where(ok, mid, lo))
        nhi = jnp.where(done, hi, jnp.where(ok, hi, mid))
        ndone = done | (cnt == K) | (nlo + 1 >= nhi)
        lo_ref[...] = nlo
        hi_ref[...] = nhi
        done_ref[...] = ndone.astype(jnp.int32)
        return jnp.sum(1 - ndone.astype(jnp.int32))

    jax.lax.while_loop(cond, body, jnp.sum(1 - done0))
    sel = jnp.where(keys >= lo_ref[...], jnp.maximum(keys, 0), 0)
    feat_ref[...] = jax.lax.bitcast_convert_type(sel, jnp.float32)


def _decode_kernel(feat_ref, w_ref, pb_ref, out_ref):
    j = pl.program_id(0)
    acc = jax.lax.dot_general(
        feat_ref[...], w_ref[...],
        (((1,), (0,)), ((), ())), preferred_element_type=jnp.float32)

    @pl.when(j == 0)
    def _():
        out_ref[...] = acc + pb_ref[...]

    @pl.when(j != 0)
    def _():
        out_ref[...] += acc


def kernel(x, W, pre_bias, latent_bias):
    pb = pre_bias.reshape(1, D)
    lb = latent_bias.reshape(1, H)

    keys = pl.pallas_call(
        _encode_kernel,
        grid=(H // _BH_ENC,),
        in_specs=[
            pl.BlockSpec((N, D), lambda j: (0, 0)),
            pl.BlockSpec((_BH_ENC, D), lambda j: (j, 0)),
            pl.BlockSpec((1, D), lambda j: (0, 0)),
            pl.BlockSpec((1, _BH_ENC), lambda j: (0, j)),
        ],
        out_specs=pl.BlockSpec((N, _BH_ENC), lambda j: (0, j)),
        out_shape=jax.ShapeDtypeStruct((N, H), jnp.int32),
    )(x, W, pb, lb)

    features = pl.pallas_call(
        _select_kernel,
        grid=(N // _BN_SEL,),
        in_specs=[pl.BlockSpec((_BN_SEL, H), lambda i: (i, 0))],
        out_specs=pl.BlockSpec((_BN_SEL, H), lambda i: (i, 0)),
        out_shape=jax.ShapeDtypeStruct((N, H), jnp.float32),
        scratch_shapes=[
            pltpu.VMEM((_BN_SEL, 1), jnp.int32),
            pltpu.VMEM((_BN_SEL, 1), jnp.int32),
            pltpu.VMEM((_BN_SEL, 1), jnp.int32),
        ],
    )(keys)

    x_hat = pl.pallas_call(
        _decode_kernel,
        grid=(H // _BH_DEC,),
        in_specs=[
            pl.BlockSpec((N, _BH_DEC), lambda j: (0, j)),
            pl.BlockSpec((_BH_DEC, D), lambda j: (j, 0)),
            pl.BlockSpec((1, D), lambda j: (0, 0)),
        ],
        out_specs=pl.BlockSpec((N, D), lambda j: (0, 0)),
        out_shape=jax.ShapeDtypeStruct((N, D), jnp.float32),
    )(features, W, pb)

    return (x_hat, features)


# speedup vs baseline: 1.0294x; 1.0294x over previous
import jax
import jax.numpy as jnp
from jax.experimental import pallas as pl
from jax.experimental.pallas import tpu as pltpu

N = 2048
D = 2048
H = 16384
K = 64

_BH_ENC = 512
_BN_SEL = 128
_BN_DEC = 1024
_BH_DEC = 1024


def _encode_kernel(x_ref, w_ref, pb_ref, lb_ref, out_ref):
    xc = x_ref[...] - pb_ref[...]
    pa = jax.lax.dot_general(
        xc, w_ref[...], (((1,), (1,)), ((), ())),
        preferred_element_type=jnp.float32) + lb_ref[...]
    b = jax.lax.bitcast_convert_type(pa, jnp.int32)
    out_ref[...] = jnp.where(b >= 0, b, b ^ jnp.int32(0x7FFFFFFF))


def _select_kernel(keys_ref, feat_ref, lo_ref, hi_ref, done_ref):
    keys = keys_ref[...]
    lo0 = jnp.min(keys, axis=1, keepdims=True)
    hi0 = jnp.max(keys, axis=1, keepdims=True) + 1
    done0 = (lo0 + 1 >= hi0).astype(jnp.int32)
    lo_ref[...] = lo0
    hi_ref[...] = hi0
    done_ref[...] = done0

    def cond(n_active):
        return n_active > 0

    def body(_):
        lo = lo_ref[...]
        hi = hi_ref[...]
        done = done_ref[...] > 0
        mid = (lo >> 1) + (hi >> 1) + (lo & hi & 1)
        cnt = jnp.sum((keys >= mid).astype(jnp.int32), axis=1, keepdims=True)
        ok = cnt >= K
        nlo = jnp.where(done, lo, jnp.where(ok, mid, lo))
        nhi = jnp.where(done, hi, jnp.where(ok, hi, mid))
        ndone = done | (cnt == K) | (nlo + 1 >= nhi)
        lo_ref[...] = nlo
        hi_ref[...] = nhi
        done_ref[...] = ndone.astype(jnp.int32)
        return jnp.sum(1 - ndone.astype(jnp.int32))

    jax.lax.while_loop(cond, body, jnp.sum(1 - done0))
    sel = jnp.where(keys >= lo_ref[...], jnp.maximum(keys, 0), 0)
    feat_ref[...] = jax.lax.bitcast_convert_type(sel, jnp.float32)


def _decode_kernel(feat_ref, w_ref, pb_ref, out_ref):
    j = pl.program_id(1)
    acc = jax.lax.dot_general(
        feat_ref[...], w_ref[...],
        (((1,), (0,)), ((), ())), preferred_element_type=jnp.float32)

    @pl.when(j == 0)
    def _():
        out_ref[...] = acc + pb_ref[...]

    @pl.when(j != 0)
    def _():
        out_ref[...] += acc


def kernel(x, W, pre_bias, latent_bias):
    pb = pre_bias.reshape(1, D)
    lb = latent_bias.reshape(1, H)

    keys = pl.pallas_call(
        _encode_kernel,
        grid=(H // _BH_ENC,),
        in_specs=[
            pl.BlockSpec((N, D), lambda j: (0, 0)),
            pl.BlockSpec((_BH_ENC, D), lambda j: (j, 0)),
            pl.BlockSpec((1, D), lambda j: (0, 0)),
            pl.BlockSpec((1, _BH_ENC), lambda j: (0, j)),
        ],
        out_specs=pl.BlockSpec((N, _BH_ENC), lambda j: (0, j)),
        out_shape=jax.ShapeDtypeStruct((N, H), jnp.int32),
    )(x, W, pb, lb)

    features = pl.pallas_call(
        _select_kernel,
        grid=(N // _BN_SEL,),
        in_specs=[pl.BlockSpec((_BN_SEL, H), lambda i: (i, 0))],
        out_specs=pl.BlockSpec((_BN_SEL, H), lambda i: (i, 0)),
        out_shape=jax.ShapeDtypeStruct((N, H), jnp.float32),
        scratch_shapes=[
            pltpu.VMEM((_BN_SEL, 1), jnp.int32),
            pltpu.VMEM((_BN_SEL, 1), jnp.int32),
            pltpu.VMEM((_BN_SEL, 1), jnp.int32),
        ],
    )(keys)

    x_hat = pl.pallas_call(
        _decode_kernel,
        grid=(N // _BN_DEC, H // _BH_DEC),
        in_specs=[
            pl.BlockSpec((_BN_DEC, _BH_DEC), lambda i, j: (i, j)),
            pl.BlockSpec((_BH_DEC, D), lambda i, j: (j, 0)),
            pl.BlockSpec((1, D), lambda i, j: (0, 0)),
        ],
        out_specs=pl.BlockSpec((_BN_DEC, D), lambda i, j: (i, 0)),
        out_shape=jax.ShapeDtypeStruct((N, D), jnp.float32),
    )(features, W, pb)

    return (x_hat, features)


# speedup vs baseline: 1.1717x; 1.1382x over previous
import jax
import jax.numpy as jnp
from jax.experimental import pallas as pl
from jax.experimental.pallas import tpu as pltpu

N = 2048
D = 2048
H = 16384
K = 64

_BH_ENC = 512
_BN_SEL = 128
_BN_DEC = 1024
_BH_DEC = 1024


def _encode_kernel(x_ref, w_ref, pb_ref, lb_ref, out_ref):
    xc = x_ref[...] - pb_ref[...]
    pa = jax.lax.dot_general(
        xc, w_ref[...], (((1,), (1,)), ((), ())),
        preferred_element_type=jnp.float32) + lb_ref[...]
    b = jax.lax.bitcast_convert_type(pa, jnp.int32)
    out_ref[...] = jnp.where(b >= 0, b, b ^ jnp.int32(0x7FFFFFFF))


def _key_of_f32(v):
    b = jax.lax.bitcast_convert_type(v, jnp.int32)
    return jnp.where(b >= 0, b, b ^ jnp.int32(0x7FFFFFFF))


def _select_kernel(keys_ref, feat_ref, lo_ref, hi_ref, done_ref):
    keys = keys_ref[...]
    kmin = jnp.min(keys, axis=1, keepdims=True)
    kmax1 = jnp.max(keys, axis=1, keepdims=True) + 1
    pa = jax.lax.bitcast_convert_type(
        jnp.where(keys >= 0, keys, keys ^ jnp.int32(0x7FFFFFFF)), jnp.float32)
    s1 = jnp.sum(pa, axis=1, keepdims=True)
    s2 = jnp.sum(pa * pa, axis=1, keepdims=True)
    mu = s1 * (1.0 / H)
    sig = jnp.sqrt(jnp.maximum(s2 * (1.0 / H) - mu * mu, 0.0))
    glo = _key_of_f32(mu + 2.1 * sig)
    ghi = _key_of_f32(mu + 3.6 * sig)
    clo = jnp.sum((keys >= glo).astype(jnp.int32), axis=1, keepdims=True)
    chi = jnp.sum((keys >= ghi).astype(jnp.int32), axis=1, keepdims=True)
    lo0 = jnp.where(clo >= K, glo, kmin)
    hi0 = jnp.where(chi < K, jnp.maximum(ghi, lo0 + 1), kmax1)
    done0 = ((clo == K) | (lo0 + 1 >= hi0)).astype(jnp.int32)
    lo_ref[...] = lo0
    hi_ref[...] = hi0
    done_ref[...] = done0

    def cond(n_active):
        return n_active > 0

    def body(_):
        lo = lo_ref[...]
        hi = hi_ref[...]
        done = done_ref[...] > 0
        mid = (lo >> 1) + (hi >> 1) + (lo & hi & 1)
        cnt = jnp.sum((keys >= mid).astype(jnp.int32), axis=1, keepdims=True)
        ok = cnt >= K
        nlo = jnp.where(done, lo, jnp.where(ok, mid, lo))
        nhi = jnp.where(done, hi, jnp.where(ok, hi, mid))
        ndone = done | (cnt == K) | (nlo + 1 >= nhi)
        lo_ref[...] = nlo
        hi_ref[...] = nhi
        done_ref[...] = ndone.astype(jnp.int32)
        return jnp.sum(1 - ndone.astype(jnp.int32))

    jax.lax.while_loop(cond, body, jnp.sum(1 - done0))
    sel = jnp.where(keys >= lo_ref[...], jnp.maximum(keys, 0), 0)
    feat_ref[...] = jax.lax.bitcast_convert_type(sel, jnp.float32)


def _decode_kernel(feat_ref, w_ref, pb_ref, out_ref):
    j = pl.program_id(1)
    acc = jax.lax.dot_general(
        feat_ref[...], w_ref[...],
        (((1,), (0,)), ((), ())), preferred_element_type=jnp.float32)

    @pl.when(j == 0)
    def _():
        out_ref[...] = acc + pb_ref[...]

    @pl.when(j != 0)
    def _():
        out_ref[...] += acc


def kernel(x, W, pre_bias, latent_bias):
    pb = pre_bias.reshape(1, D)
    lb = latent_bias.reshape(1, H)

    keys = pl.pallas_call(
        _encode_kernel,
        grid=(H // _BH_ENC,),
        in_specs=[
            pl.BlockSpec((N, D), lambda j: (0, 0)),
            pl.BlockSpec((_BH_ENC, D), lambda j: (j, 0)),
            pl.BlockSpec((1, D), lambda j: (0, 0)),
            pl.BlockSpec((1, _BH_ENC), lambda j: (0, j)),
        ],
        out_specs=pl.BlockSpec((N, _BH_ENC), lambda j: (0, j)),
        out_shape=jax.ShapeDtypeStruct((N, H), jnp.int32),
    )(x, W, pb, lb)

    features = pl.pallas_call(
        _select_kernel,
        grid=(N // _BN_SEL,),
        in_specs=[pl.BlockSpec((_BN_SEL, H), lambda i: (i, 0))],
        out_specs=pl.BlockSpec((_BN_SEL, H), lambda i: (i, 0)),
        out_shape=jax.ShapeDtypeStruct((N, H), jnp.float32),
        scratch_shapes=[
            pltpu.VMEM((_BN_SEL, 1), jnp.int32),
            pltpu.VMEM((_BN_SEL, 1), jnp.int32),
            pltpu.VMEM((_BN_SEL, 1), jnp.int32),
        ],
    )(keys)

    x_hat = pl.pallas_call(
        _decode_kernel,
        grid=(N // _BN_DEC, H // _BH_DEC),
        in_specs=[
            pl.BlockSpec((_BN_DEC, _BH_DEC), lambda i, j: (i, j)),
            pl.BlockSpec((_BH_DEC, D), lambda i, j: (j, 0)),
            pl.BlockSpec((1, D), lambda i, j: (0, 0)),
        ],
        out_specs=pl.BlockSpec((_BN_DEC, D), lambda i, j: (i, 0)),
        out_shape=jax.ShapeDtypeStruct((N, D), jnp.float32),
    )(features, W, pb)

    return (x_hat, features)
